# baseline (speedup 1.0000x reference)
import functools

import jax
import jax.numpy as jnp
from jax import lax
from jax.experimental import pallas as pl
from jax.experimental.pallas import tpu as pltpu
from jax.experimental.pallas import tpu_sc as plsc

D_MODEL = 4096
NUM_EXPERTS = 64
TOP_K = 8
CAPACITY_FACTOR = 1.25
ALPHA = 0.01

BLK = 1024
N_TOK = 16384
NW = 32
TPW = N_TOK // NW
L = 16


def _logits_kernel(x_ref, wt_ref, b_ref, out_ref):
    out_ref[...] = jnp.dot(x_ref[...], wt_ref[...],
                           preferred_element_type=jnp.float32) + b_ref[...]


def _merge_desc(ak, av, bk, bv):
    rbk = lax.rev(bk, (0,))
    rbv = lax.rev(bv, (0,))
    ge = ak >= rbk
    hk = jnp.where(ge, ak, rbk)
    hv = jnp.where(ge, av, rbv)
    return plsc.sort_key_val(hk, hv, descending=True)


def _route_sc_body(lg_hbm, idx_hbm, w_hbm, load_hbm, lg_v, idx_v, w_v,
                   load_v, sem):
    wid = lax.axis_index("s") * 2 + lax.axis_index("c")
    base = wid * TPW
    pltpu.async_copy(lg_hbm.at[pl.ds(base, TPW)], lg_v, sem).wait()

    lane = jnp.arange(L, dtype=jnp.int32)
    mask8 = lane < TOP_K
    ones8 = jnp.where(mask8, jnp.float32(1.0), jnp.float32(0.0))
    zeros = jnp.zeros((L,), jnp.float32)
    for j in range(NUM_EXPERTS // L):
        load_v[pl.ds(j * L, L)] = zeros

    def body(t, carry):
        ks = []
        vs = []
        for j in range(NUM_EXPERTS // L):
            kj = lg_v[t, pl.ds(j * L, L)]
            vj = lane + jnp.int32(j * L)
            kjs, vjs = plsc.sort_key_val(kj, vj, descending=True)
            ks.append(kjs)
            vs.append(vjs)
        k01, v01 = _merge_desc(ks[0], vs[0], ks[1], vs[1])
        k23, v23 = _merge_desc(ks[2], vs[2], ks[3], vs[3])
        kf, vf = _merge_desc(k01, v01, k23, v23)

        m = jnp.max(kf)
        e = jnp.where(mask8, jnp.exp(kf - m), 0.0)
        w = e / jnp.sum(e)
        idx_v[t, :] = vf
        w_v[t, :] = w
        plsc.addupdate_scatter(load_v, [vf], ones8, mask=mask8)
        return carry

    lax.fori_loop(0, TPW, body, 0)

    pltpu.sync_copy(idx_v, idx_hbm.at[pl.ds(base, TPW)])
    pltpu.sync_copy(w_v, w_hbm.at[pl.ds(base, TPW)])
    pltpu.sync_copy(load_v, load_hbm.at[wid])


def kernel(x, W, b):
    batch, seq, d_model = x.shape
    n_tokens = batch * seq
    xf = x.reshape(n_tokens, d_model)
    wt = W.T
    n_steps = n_tokens // BLK

    logits = pl.pallas_call(
        _logits_kernel,
        grid=(n_steps,),
        in_specs=[
            pl.BlockSpec((BLK, d_model), lambda i: (i, 0)),
            pl.BlockSpec((d_model, NUM_EXPERTS), lambda i: (0, 0)),
            pl.BlockSpec((NUM_EXPERTS,), lambda i: (0,)),
        ],
        out_specs=pl.BlockSpec((BLK, NUM_EXPERTS), lambda i: (i, 0)),
        out_shape=jax.ShapeDtypeStruct((n_tokens, NUM_EXPERTS), jnp.float32),
    )(xf, wt, b)

    route = functools.partial(
        pl.kernel,
        mesh=plsc.VectorSubcoreMesh(core_axis_name="c", subcore_axis_name="s"),
        compiler_params=pltpu.CompilerParams(needs_layout_passes=False,
                                             use_tc_tiling_on_sc=False),
        out_type=[
            jax.ShapeDtypeStruct((N_TOK, L), jnp.int32),
            jax.ShapeDtypeStruct((N_TOK, L), jnp.float32),
            jax.ShapeDtypeStruct((NW, NUM_EXPERTS), jnp.float32),
        ],
        scratch_types=[
            pltpu.VMEM((TPW, NUM_EXPERTS), jnp.float32),
            pltpu.VMEM((TPW, L), jnp.int32),
            pltpu.VMEM((TPW, L), jnp.float32),
            pltpu.VMEM((NUM_EXPERTS,), jnp.float32),
            pltpu.SemaphoreType.DMA,
        ],
    )(_route_sc_body)

    idx16, w16, loads = route(logits)

    load = loads.sum(axis=0)
    capacity = CAPACITY_FACTOR * (n_tokens * TOP_K) / NUM_EXPERTS
    aux = ALPHA * jax.nn.relu(load - capacity).sum() / NUM_EXPERTS / n_tokens

    return (idx16[:, :TOP_K].reshape(batch, seq, TOP_K),
            w16[:, :TOP_K].reshape(batch, seq, TOP_K),
            aux)


# speedup vs baseline: 1.7312x; 1.7312x over previous
import functools

import jax
import jax.numpy as jnp
from jax import lax
from jax.experimental import pallas as pl
from jax.experimental.pallas import tpu as pltpu

D_MODEL = 4096
NUM_EXPERTS = 64
TOP_K = 8
CAPACITY_FACTOR = 1.25
ALPHA = 0.01

BLK = 1024


def _gate_kernel(x_ref, wt_ref, b_ref, idx_ref, w_ref, aux_ref, load_acc,
                 *, n_steps, n_tokens):
    i = pl.program_id(0)

    logits = lax.dot_general(x_ref[...], wt_ref[...],
                             (((1,), (1,)), ((), ())),
                             preferred_element_type=jnp.float32) + b_ref[...]

    lt = logits.T
    rowi = lax.broadcasted_iota(jnp.int32, (NUM_EXPERTS, BLK), 0)
    neg = jnp.float32(-1e30)

    idx_rows = []
    val_rows = []
    pm = lt
    for _ in range(TOP_K):
        mv = jnp.max(pm, axis=0, keepdims=True)
        is_max = pm == mv
        sel = jnp.min(jnp.where(is_max, rowi, NUM_EXPERTS), axis=0,
                      keepdims=True)
        idx_rows.append(sel)
        val_rows.append(mv)
        pm = jnp.where(rowi == sel, neg, pm)

    idx_t = jnp.concatenate(idx_rows, axis=0)
    val_t = jnp.concatenate(val_rows, axis=0)
    e = jnp.exp(val_t - val_t[0:1])
    w_t = e / jnp.sum(e, axis=0, keepdims=True)
    idx_ref[...] = idx_t.T
    w_ref[...] = w_t.T

    sel_mask = (pm <= neg).astype(jnp.float32)
    load_part = jnp.sum(sel_mask, axis=1, keepdims=True)

    @pl.when(i == 0)
    def _init():
        load_acc[...] = jnp.zeros_like(load_acc)

    load_acc[...] += load_part

    @pl.when(i == n_steps - 1)
    def _finalize():
        load = load_acc[...]
        capacity = CAPACITY_FACTOR * (n_tokens * TOP_K) / NUM_EXPERTS
        penalty = jnp.sum(jnp.maximum(load - capacity, 0.0))
        aux = ALPHA * penalty / NUM_EXPERTS / n_tokens
        aux_ref[...] = aux.reshape(1, 1)


def kernel(x, W, b):
    batch, seq, d_model = x.shape
    n_tokens = batch * seq
    xf = x.reshape(n_tokens, d_model)
    wt = W
    n_steps = n_tokens // BLK

    idx, w, aux = pl.pallas_call(
        functools.partial(_gate_kernel, n_steps=n_steps, n_tokens=n_tokens),
        grid=(n_steps,),
        in_specs=[
            pl.BlockSpec((BLK, d_model), lambda i: (i, 0)),
            pl.BlockSpec((NUM_EXPERTS, d_model), lambda i: (0, 0)),
            pl.BlockSpec((NUM_EXPERTS,), lambda i: (0,)),
        ],
        out_specs=[
            pl.BlockSpec((BLK, TOP_K), lambda i: (i, 0)),
            pl.BlockSpec((BLK, TOP_K), lambda i: (i, 0)),
            pl.BlockSpec((1, 1), lambda i: (0, 0)),
        ],
        out_shape=[
            jax.ShapeDtypeStruct((n_tokens, TOP_K), jnp.int32),
            jax.ShapeDtypeStruct((n_tokens, TOP_K), jnp.float32),
            jax.ShapeDtypeStruct((1, 1), jnp.float32),
        ],
        scratch_shapes=[pltpu.VMEM((NUM_EXPERTS, 1), jnp.float32)],
    )(xf, wt, b)

    return (idx.reshape(batch, seq, TOP_K),
            w.reshape(batch, seq, TOP_K),
            aux[0, 0])
